# baseline (speedup 1.0000x reference)
import functools

import jax
import jax.numpy as jnp
from jax import lax
from jax.experimental import pallas as pl
from jax.experimental.pallas import tpu as pltpu
from jax.experimental.pallas import tpu_sc as plsc

_NODES = 4096
_HID = 512
_NSC_CORES = 2
_NSUB = 16
_SC_GRAPHS = _NSC_CORES
_RPS = _NODES // _NSUB
_CHUNK = 64
_NCHUNK = _RPS // _CHUNK
_NV = _HID // 16
_INV_N = 1.0 / _NODES


def _tc_block(h_ref, w_ref, b_ref, ms_ref, out_ref):
    x = h_ref[:, :]
    m1 = jnp.mean(x, axis=0, keepdims=True)
    m2 = jnp.mean(x * x, axis=0, keepdims=True)
    s = ms_ref[:, :]
    var = m2 - (2.0 * s - s * s) * (m1 * m1)
    inv = jax.lax.rsqrt(var + 1e-6)
    a = w_ref[:, :] * inv
    out_ref[:, :] = a * x + (b_ref[:, :] - a * (s * m1))


def _rsqrt_nr(x):
    i = plsc.bitcast(x, jnp.int32)
    magic = jnp.full((16,), 0x5F3759DF, jnp.int32)
    y = plsc.bitcast(magic - (i >> 1), jnp.float32)
    half = 0.5 * x
    for _ in range(3):
        y = y * (1.5 - half * y * y)
    return y


def _sc_body(h_hbm, w_hbm, b_hbm, ms_hbm, out_hbm,
             buf, acc, wv, bv, msv_r, ab, allacc, shared, sem):
    c = lax.axis_index("c")
    s = lax.axis_index("s")
    row0 = c * _NODES + s * _RPS

    pltpu.sync_copy(w_hbm, wv)
    pltpu.sync_copy(b_hbm, bv)
    pltpu.sync_copy(ms_hbm, msv_r)

    cps = [
        pltpu.make_async_copy(
            h_hbm.at[pl.ds(row0 + ch * _CHUNK, _CHUNK), :],
            buf.at[ch % 2],
            sem,
        )
        for ch in range(_NCHUNK)
    ]
    cps[0].start()
    for ch in range(_NCHUNK):
        cps[ch].wait()
        if ch + 1 < _NCHUNK:
            cps[ch + 1].start()
        bref = buf.at[ch % 2]
        for j in range(_NV):
            def body(r, carry):
                sv, qv = carry
                v = bref[r, pl.ds(16 * j, 16)]
                return sv + v, qv + v * v
            z = jnp.zeros((16,), jnp.float32)
            sv, qv = lax.fori_loop(0, _CHUNK, body, (z, z))
            if ch == 0:
                acc[0, pl.ds(16 * j, 16)] = sv
                acc[1, pl.ds(16 * j, 16)] = qv
            else:
                acc[0, pl.ds(16 * j, 16)] += sv
                acc[1, pl.ds(16 * j, 16)] += qv

    pltpu.sync_copy(acc, shared.at[s])
    plsc.subcore_barrier()
    pltpu.sync_copy(shared, allacc)
    for j in range(_NV):
        sv = jnp.zeros((16,), jnp.float32)
        qv = jnp.zeros((16,), jnp.float32)
        for p in range(_NSUB):
            sv = sv + allacc[p, 0, pl.ds(16 * j, 16)]
            qv = qv + allacc[p, 1, pl.ds(16 * j, 16)]
        m1 = sv * _INV_N
        m2 = qv * _INV_N
        msv = msv_r[pl.ds(16 * j, 16)]
        var = m2 - (2.0 * msv - msv * msv) * (m1 * m1)
        inv = _rsqrt_nr(var + 1e-6)
        a = wv[pl.ds(16 * j, 16)] * inv
        ab[0, pl.ds(16 * j, 16)] = a
        ab[1, pl.ds(16 * j, 16)] = bv[pl.ds(16 * j, 16)] - a * (msv * m1)

    cps2 = [
        pltpu.make_async_copy(
            h_hbm.at[pl.ds(row0 + ch * _CHUNK, _CHUNK), :],
            buf.at[ch % 2],
            sem,
        )
        for ch in range(_NCHUNK)
    ]
    cps2[0].start()
    for ch in range(_NCHUNK):
        cps2[ch].wait()
        bref = buf.at[ch % 2]
        for j in range(_NV):
            a = ab[0, pl.ds(16 * j, 16)]
            bb = ab[1, pl.ds(16 * j, 16)]
            def body2(r, carry):
                bref[r, pl.ds(16 * j, 16)] = a * bref[r, pl.ds(16 * j, 16)] + bb
                return carry
            lax.fori_loop(0, _CHUNK, body2, 0)
        pltpu.sync_copy(
            buf.at[ch % 2],
            out_hbm.at[pl.ds(row0 + ch * _CHUNK, _CHUNK), :],
        )
        if ch + 1 < _NCHUNK:
            cps2[ch + 1].start()


def _sc_norm(h_tail, weight, bias, mean_scale):
    mesh = plsc.VectorSubcoreMesh(
        core_axis_name="c", subcore_axis_name="s",
        num_cores=_NSC_CORES, num_subcores=_NSUB,
    )
    return pl.kernel(
        _sc_body,
        mesh=mesh,
        compiler_params=pltpu.CompilerParams(needs_layout_passes=False),
        out_type=jax.ShapeDtypeStruct((_SC_GRAPHS * _NODES, _HID), jnp.float32),
        scratch_types=[
            pltpu.VMEM((2, _CHUNK, _HID), jnp.float32),
            pltpu.VMEM((2, _HID), jnp.float32),
            pltpu.VMEM((_HID,), jnp.float32),
            pltpu.VMEM((_HID,), jnp.float32),
            pltpu.VMEM((_HID,), jnp.float32),
            pltpu.VMEM((2, _HID), jnp.float32),
            pltpu.VMEM((_NSUB, 2, _HID), jnp.float32),
            pltpu.VMEM_SHARED((_NSUB, 2, _HID), jnp.float32),
            pltpu.SemaphoreType.DMA,
        ],
    )(h_tail, weight, bias, mean_scale)


@jax.jit
def kernel(h, weight, bias, mean_scale):
    rows, hidden = h.shape
    batch = rows // _NODES
    tc_graphs = batch - _SC_GRAPHS
    tc_rows = tc_graphs * _NODES
    w2 = weight.reshape(1, hidden)
    b2 = bias.reshape(1, hidden)
    ms2 = mean_scale.reshape(1, hidden)
    out_full = pl.pallas_call(
        _tc_block,
        grid=(tc_graphs,),
        in_specs=[
            pl.BlockSpec((_NODES, hidden), lambda i: (i, 0)),
            pl.BlockSpec((1, hidden), lambda i: (0, 0)),
            pl.BlockSpec((1, hidden), lambda i: (0, 0)),
            pl.BlockSpec((1, hidden), lambda i: (0, 0)),
        ],
        out_specs=pl.BlockSpec((_NODES, hidden), lambda i: (i, 0)),
        out_shape=jax.ShapeDtypeStruct((rows, hidden), h.dtype),
    )(h, w2, b2, ms2)
    out_sc = _sc_norm(h[tc_rows:], weight, bias, mean_scale)
    return lax.dynamic_update_slice(out_full, out_sc, (tc_rows, 0))


# speedup vs baseline: 1.9867x; 1.9867x over previous
import jax
import jax.numpy as jnp
from jax.experimental import pallas as pl

_NODES = 4096


def _graphnorm_block(h_ref, w_ref, b_ref, ms_ref, out_ref):
    x = h_ref[:, :]
    m1 = jnp.mean(x, axis=0, keepdims=True)
    m2 = jnp.mean(x * x, axis=0, keepdims=True)
    s = ms_ref[:, :]
    var = m2 - (2.0 * s - s * s) * (m1 * m1)
    inv = jax.lax.rsqrt(var + 1e-6)
    a = w_ref[:, :] * inv
    out_ref[:, :] = a * x + (b_ref[:, :] - a * (s * m1))


@jax.jit
def kernel(h, weight, bias, mean_scale):
    rows, hidden = h.shape
    batch = rows // _NODES
    w2 = weight.reshape(1, hidden)
    b2 = bias.reshape(1, hidden)
    ms2 = mean_scale.reshape(1, hidden)
    out = pl.pallas_call(
        _graphnorm_block,
        grid=(batch,),
        in_specs=[
            pl.BlockSpec((_NODES, hidden), lambda i: (i, 0)),
            pl.BlockSpec((1, hidden), lambda i: (0, 0)),
            pl.BlockSpec((1, hidden), lambda i: (0, 0)),
            pl.BlockSpec((1, hidden), lambda i: (0, 0)),
        ],
        out_specs=pl.BlockSpec((_NODES, hidden), lambda i: (i, 0)),
        out_shape=jax.ShapeDtypeStruct((rows, hidden), h.dtype),
    )(h, w2, b2, ms2)
    return out
